# baseline (speedup 1.0000x reference)
import functools

import jax
import jax.numpy as jnp
from jax import lax
from jax.experimental import pallas as pl
from jax.experimental.pallas import tpu as pltpu
from jax.experimental.pallas import tpu_sc as plsc

BATCH = 4096
SEQ = 200
D_MODEL = 64
NUM_CORES = 2
NUM_SUBCORES = 16
NUM_TILES = NUM_CORES * NUM_SUBCORES
ITEMS_PER_TILE = BATCH // NUM_TILES
HALF = SEQ // 2
OUT_WORDS = D_MODEL * SEQ
N_DC = D_MODEL // 16


def _sc_embed_permute(x3, table):
    mesh = plsc.VectorSubcoreMesh(core_axis_name="c", subcore_axis_name="s")

    @functools.partial(
        pl.kernel,
        mesh=mesh,
        compiler_params=pltpu.CompilerParams(
            needs_layout_passes=False, use_tc_tiling_on_sc=False
        ),
        out_type=jax.ShapeDtypeStruct((BATCH, OUT_WORDS), jnp.float32),
        scratch_types=[
            pltpu.VMEM((ITEMS_PER_TILE, 2, HALF), jnp.int32),
            pltpu.VMEM((SEQ, D_MODEL), jnp.float32),
            pltpu.VMEM((OUT_WORDS,), jnp.float32),
            pltpu.SemaphoreType.DMA,
        ],
    )
    def k(x_hbm, table_hbm, out_hbm, idx_all, rows_v, out_v, sem):
        wid = lax.axis_index("s") * NUM_CORES + lax.axis_index("c")
        base = wid * ITEMS_PER_TILE
        pltpu.sync_copy(x_hbm.at[pl.ds(base, ITEMS_PER_TILE)], idx_all)

        iota = lax.iota(jnp.int32, 16)

        def body(i, carry):
            c0 = pltpu.async_copy(
                table_hbm.at[idx_all.at[i, 0]], rows_v.at[pl.ds(0, HALF)], sem
            )
            c1 = pltpu.async_copy(
                table_hbm.at[idx_all.at[i, 1]], rows_v.at[pl.ds(HALF, HALF)], sem
            )
            c0.wait()
            c1.wait()

            dvecs = [(iota + dc * 16) * SEQ for dc in range(N_DC)]

            def lbody(l, carry2):
                for dc in range(N_DC):
                    vals = rows_v[l, pl.ds(dc * 16, 16)]
                    plsc.store_scatter(out_v, [dvecs[dc] + l], vals)
                return carry2

            lax.fori_loop(0, SEQ, lbody, 0)
            pltpu.sync_copy(out_v, out_hbm.at[base + i])
            return carry

        lax.fori_loop(0, ITEMS_PER_TILE, body, 0)

    return k(x3, table)


def kernel(x, table):
    x3 = x.astype(jnp.int32).reshape(BATCH, 2, HALF)
    out = _sc_embed_permute(x3, table)
    return out.reshape(BATCH, D_MODEL, SEQ)


# speedup vs baseline: 1.1763x; 1.1763x over previous
import functools

import jax
import jax.numpy as jnp
from jax import lax
from jax.experimental import pallas as pl
from jax.experimental.pallas import tpu as pltpu
from jax.experimental.pallas import tpu_sc as plsc

BATCH = 4096
SEQ = 200
D_MODEL = 64
NUM_CORES = 2
NUM_SUBCORES = 16
NUM_TILES = NUM_CORES * NUM_SUBCORES
ITEMS_PER_TILE = BATCH // NUM_TILES
HALF = SEQ // 2
OUT_WORDS = D_MODEL * SEQ
N_DC = D_MODEL // 16
L_UNROLL = 4


def _sc_embed_permute(x3, table):
    mesh = plsc.VectorSubcoreMesh(core_axis_name="c", subcore_axis_name="s")

    @functools.partial(
        pl.kernel,
        mesh=mesh,
        compiler_params=pltpu.CompilerParams(
            needs_layout_passes=False, use_tc_tiling_on_sc=False
        ),
        out_type=jax.ShapeDtypeStruct((BATCH, OUT_WORDS), jnp.float32),
        scratch_types=[
            pltpu.VMEM((ITEMS_PER_TILE, 2, HALF), jnp.int32),
            pltpu.VMEM((SEQ, D_MODEL), jnp.float32),
            pltpu.VMEM((SEQ, D_MODEL), jnp.float32),
            pltpu.VMEM((OUT_WORDS,), jnp.float32),
            pltpu.VMEM((OUT_WORDS,), jnp.float32),
            pltpu.SemaphoreType.DMA,
            pltpu.SemaphoreType.DMA,
            pltpu.SemaphoreType.DMA,
            pltpu.SemaphoreType.DMA,
        ],
    )
    def k(x_hbm, table_hbm, out_hbm, idx_all, rows0, rows1, out0, out1,
          sg0, sg1, sw0, sw1):
        wid = lax.axis_index("s") * NUM_CORES + lax.axis_index("c")
        base = wid * ITEMS_PER_TILE
        pltpu.sync_copy(x_hbm.at[pl.ds(base, ITEMS_PER_TILE)], idx_all)

        iota = lax.iota(jnp.int32, 16)
        dvecs = [(iota + dc * 16) * SEQ for dc in range(N_DC)]

        def start_gather(i, rows, sem):
            pltpu.async_copy(
                table_hbm.at[idx_all.at[i, 0]], rows.at[pl.ds(0, HALF)], sem
            )
            pltpu.async_copy(
                table_hbm.at[idx_all.at[i, 1]], rows.at[pl.ds(HALF, HALF)], sem
            )

        def wait_gather(rows, sem):
            pltpu.make_async_copy(
                table_hbm.at[pl.ds(0, SEQ)], rows, sem
            ).wait()

        def wait_write(out_v, sem):
            pltpu.make_async_copy(out_v, out_hbm.at[base], sem).wait()

        def transpose(rows, out_v):
            def lbody(l, carry2):
                for u in range(L_UNROLL):
                    li = l * L_UNROLL + u
                    lsplat = jnp.full((16,), 0, jnp.int32) + li
                    for dc in range(N_DC):
                        vals = rows[li, pl.ds(dc * 16, 16)]
                        plsc.store_scatter(out_v, [dvecs[dc] + lsplat], vals)
                return carry2

            lax.fori_loop(0, SEQ // L_UNROLL, lbody, 0)

        start_gather(0, rows0, sg0)
        start_gather(1, rows1, sg1)

        def body(k_, carry):
            def side(i, rows, out_v, sg, sw):
                wait_gather(rows, sg)

                @pl.when(k_ > 0)
                def _():
                    wait_write(out_v, sw)

                transpose(rows, out_v)
                pltpu.async_copy(out_v, out_hbm.at[base + i], sw)
                nxt = jnp.minimum(i + 2, ITEMS_PER_TILE - 1)
                start_gather(nxt, rows, sg)

            side(2 * k_, rows0, out0, sg0, sw0)
            side(2 * k_ + 1, rows1, out1, sg1, sw1)
            return carry

        lax.fori_loop(0, ITEMS_PER_TILE // 2, body, 0)

        wait_gather(rows0, sg0)
        wait_gather(rows1, sg1)
        wait_write(out0, sw0)
        wait_write(out1, sw1)

    return k(x3, table)


def kernel(x, table):
    x3 = x.astype(jnp.int32).reshape(BATCH, 2, HALF)
    out = _sc_embed_permute(x3, table)
    return out.reshape(BATCH, D_MODEL, SEQ)
